# baseline (speedup 1.0000x reference)
import jax
import jax.numpy as jnp
from jax import lax
from jax.experimental import pallas as pl
from jax.experimental.pallas import tpu as pltpu
from jax.experimental.pallas import tpu_sc as plsc

R = 128
N = 32768
L = 16
BLK = 32
NBLK = N // (BLK * L)
NSUM = N // (BLK * L) * L
ROWS_PER_TEC = 4
NEG = float("-inf")


def _merge_top16(tv, ti, v, vidx):
    sv, si = plsc.sort_key_val(v, vidx, descending=True)
    brv = lax.rev(sv, (0,))
    bri = lax.rev(si, (0,))
    keep = tv >= brv
    cv = jnp.where(keep, tv, brv)
    ci = jnp.where(keep, ti, bri)
    nv, ni = plsc.sort_key_val(cv, ci, descending=True)
    return nv, ni


def _row_topk(buf, summary, flagbuf, iota, four):
    @plsc.parallel_loop(0, NBLK, 1, unroll=2)
    def _(j):
        base = j * (BLK * L)
        vs = [buf[pl.ds(base + i * L, L)] for i in range(BLK)]
        while len(vs) > 1:
            nxt = [jnp.maximum(vs[2 * k], vs[2 * k + 1])
                   for k in range(len(vs) // 2)]
            if len(vs) % 2:
                nxt.append(vs[-1])
            vs = nxt
        summary[pl.ds(j * L, L)] = vs[0]

    def lmax(j, lane_max):
        base = j * (16 * L)
        vs = [summary[pl.ds(base + i * L, L)] for i in range(16)]
        while len(vs) > 1:
            vs = [jnp.maximum(vs[2 * k], vs[2 * k + 1])
                  for k in range(len(vs) // 2)] + (
                      [vs[-1]] if len(vs) % 2 else [])
        return jnp.maximum(lane_max, vs[0])

    lane_max = lax.fori_loop(
        0, NSUM // (16 * L), lmax, jnp.full((L,), NEG, jnp.float32))

    s, _ = plsc.sort_key_val(lane_max, iota, descending=True)
    thresh = lax.gather(
        s, four[:, None],
        lax.GatherDimensionNumbers(
            offset_dims=(), collapsed_slice_dims=(0,), start_index_map=(0,)),
        slice_sizes=(1,),
        mode=lax.GatherScatterMode.PROMISE_IN_BOUNDS)

    def flags_half(h):
        def fb(k, acc):
            for u in range(4):
                kk = k * 4 + u
                s_k = summary[pl.ds((h * 32 + kk) * L, L)]
                hit = s_k >= thresh
                pc = plsc.all_reduce_population_count(hit)
                bit = jnp.minimum(pc, 1)
                acc = acc | jnp.left_shift(bit, kk)
            return acc
        return lax.fori_loop(0, 8, fb, jnp.zeros((L,), jnp.int32))

    flagbuf[pl.ds(0, L)] = flags_half(0)
    flagbuf[pl.ds(L, L)] = flags_half(1)
    b_lo = flagbuf[pl.ds(0, L)][0]
    b_hi = flagbuf[pl.ds(L, L)][0]

    def scan_block(j, c):
            tv, ti = c
            base = j * (BLK * L)
            NCH = 4
            ms, ams, cnts = [], [], []
            for c0 in range(NCH):
                m = buf[pl.ds(base + c0 * L, L)]
                am = jnp.full((L,), c0, jnp.int32)
                cnt = (m >= thresh).astype(jnp.int32)
                for i in range(c0 + NCH, BLK, NCH):
                    v = buf[pl.ds(base + i * L, L)]
                    gt = v > m
                    m = jnp.where(gt, v, m)
                    am = jnp.where(gt, jnp.int32(i), am)
                    cnt = cnt + (v >= thresh).astype(jnp.int32)
                ms.append(m)
                ams.append(am)
                cnts.append(cnt)
            m, am, cnt = ms[0], ams[0], cnts[0]
            for c0 in range(1, NCH):
                gt = ms[c0] > m
                lower = (ms[c0] == m) & (ams[c0] < am)
                take = gt | lower
                m = jnp.where(take, ms[c0], m)
                am = jnp.where(take, ams[c0], am)
                cnt = cnt + cnts[c0]

            def fast(c):
                tv, ti = c
                return _merge_top16(tv, ti, m, base + am * L + iota)

            def slow(c):
                def sb(i, c):
                    tv, ti = c
                    v = buf[pl.ds(base + i * L, L)]
                    vidx = iota + (base + i * L)
                    return lax.cond(
                        jnp.any(v >= thresh),
                        lambda tv, ti: _merge_top16(tv, ti, v, vidx),
                        lambda tv, ti: (tv, ti),
                        tv, ti)
                return lax.fori_loop(0, BLK, sb, c)

            return lax.cond(jnp.any(cnt >= 2), slow, fast, (tv, ti))

    def proc_word(word, off, c):
        def wcond(st):
            return st[0] != 0

        def wbody(st):
            b, tv, ti = st
            low = b & (-b)
            lowf = lax.convert_element_type(low, jnp.float32)
            j = lax.shift_right_logical(
                lax.bitcast_convert_type(lowf, jnp.int32), 23) - 127
            j = jnp.where(low == jnp.int32(-2147483648), jnp.int32(31), j)
            tv, ti = scan_block(j + off, (tv, ti))
            return (b & (b - 1), tv, ti)

        _, tv, ti = lax.while_loop(wcond, wbody, (word,) + c)
        return tv, ti

    c = (jnp.full((L,), NEG, jnp.float32), jnp.zeros((L,), jnp.int32))
    c = proc_word(b_lo, 0, c)
    return proc_word(b_hi, 32, c)


def _body(dist, packed, buf0, buf1, summary, flagbuf, outbuf, sem0, sem1):
    wid = lax.axis_index("s") * 2 + lax.axis_index("c")
    row0 = wid * ROWS_PER_TEC
    iota = lax.iota(jnp.int32, L)
    four = jnp.full((L,), 4, jnp.int32)

    bufs = (buf0, buf1)
    sems = (sem0, sem1)
    copies = [pltpu.async_copy(dist.at[row0], buf0, sem0), None]
    for r in range(ROWS_PER_TEC):
        b = r % 2
        copies[b].wait()
        if r + 1 < ROWS_PER_TEC:
            nb = (r + 1) % 2
            copies[nb] = pltpu.async_copy(
                dist.at[row0 + (r + 1)], bufs[nb], sems[nb])
        tv, ti = _row_topk(bufs[b], summary, flagbuf, iota, four)
        outbuf[pl.ds(0, L)] = plsc.bitcast(tv, jnp.int32)
        outbuf[pl.ds(L, L)] = ti
        pltpu.sync_copy(outbuf, packed.at[row0 + r])


def kernel(distance):
    mesh = plsc.VectorSubcoreMesh(core_axis_name="c", subcore_axis_name="s")
    f = pl.kernel(
        _body,
        out_type=jax.ShapeDtypeStruct((R, 2 * L), jnp.int32),
        mesh=mesh,
        compiler_params=pltpu.CompilerParams(needs_layout_passes=False),
        scratch_types=[
            pltpu.VMEM((N,), jnp.float32),
            pltpu.VMEM((N,), jnp.float32),
            pltpu.VMEM((NSUM,), jnp.float32),
            pltpu.VMEM((2 * L,), jnp.int32),
            pltpu.VMEM((2 * L,), jnp.int32),
            pltpu.SemaphoreType.DMA,
            pltpu.SemaphoreType.DMA,
        ],
    )
    packed = f(distance)
    vals = jax.lax.bitcast_convert_type(packed[:, :5], jnp.float32)
    return vals, packed[:, L:L + 5]
